# baseline (speedup 1.0000x reference)
import functools

import jax
import jax.numpy as jnp
from jax import lax
from jax.experimental import pallas as pl
from jax.experimental.pallas import tpu as pltpu
from jax.experimental.pallas import tpu_sc as plsc

_N = 10000
_NP = 10240
_E = 320000
_G = 128
_TILES = 16
_BN = 640
_NB = _NP // _BN
_R = 5
_C = 40


def _edge_pipeline(table_hbm, src_hbm, src_base, dst_hbm, dst_base, acc,
                   srcall, dstbs, rowsbs, gs, ss, ds, epc):
    n = epc // _C
    nmac = n // _R
    pltpu.sync_copy(src_hbm.at[pl.ds(src_base, epc)], srcall)

    def src_sl(k):
        return srcall.at[pl.ds(k * _C, _C)]

    def g_start(k, p):
        pltpu.async_copy(table_hbm.at[src_sl(k)], rowsbs[p], gs[p])

    def g_wait(k, p):
        pltpu.make_async_copy(table_hbm.at[src_sl(k)], rowsbs[p], gs[p]).wait()

    def d_start(k, p):
        pltpu.async_copy(dst_hbm.at[pl.ds(dst_base + k * _C, _C)], dstbs[p], ds[p])

    def d_wait(k, p):
        pltpu.make_async_copy(dst_hbm.at[pl.ds(dst_base + k * _C, _C)],
                              dstbs[p], ds[p]).wait()

    def s_start(p):
        pltpu.async_copy(rowsbs[p], acc.at[dstbs[p]], ss[p], add=True)

    def s_wait(p):
        pltpu.make_async_copy(rowsbs[p], acc.at[dstbs[p]], ss[p]).wait()

    for j in range(_R - 1):
        d_start(j, j)
        g_start(j, j)

    def body(m, carry):
        k0 = m * _R
        for t in range(_R):
            k = k0 + t
            q = (t + _R - 1) % _R
            g_wait(k, t)
            d_wait(k, t)
            s_start(t)
            if t == 0:
                @pl.when(m > 0)
                def _():
                    s_wait(q)
            else:
                s_wait(q)

            @pl.when(k + _R - 1 < n)
            def _():
                d_start(k + _R - 1, q)
                g_start(k + _R - 1, q)

        return carry

    lax.fori_loop(0, nmac, body, 0)
    s_wait((_R - 1) % _R)


def _sc_scratch(epc):
    return ([pltpu.VMEM((epc,), jnp.int32)]
            + [pltpu.VMEM((_C,), jnp.int32) for _ in range(_R)]
            + [pltpu.VMEM((_C, 128), jnp.float32) for _ in range(_R)]
            + [pltpu.VMEM_SHARED((_NP, 128), jnp.float32)]
            + [pltpu.SemaphoreType.DMA for _ in range(3 * _R)])


def _unpack_scr(scr):
    dstbs = scr[:_R]
    rowsbs = scr[_R:2 * _R]
    acc = scr[2 * _R]
    gs = scr[2 * _R + 1:3 * _R + 1]
    ss = scr[3 * _R + 1:4 * _R + 1]
    ds = scr[4 * _R + 1:5 * _R + 1]
    return dstbs, rowsbs, acc, gs, ss, ds


def _make_sc_agg_feat(goff):
    mesh = plsc.VectorSubcoreMesh(core_axis_name="c", subcore_axis_name="s")
    epc = _E // _TILES
    rows_pt = _NP // _TILES

    @functools.partial(
        pl.kernel,
        mesh=mesh,
        out_type=jax.ShapeDtypeStruct((2 * _NP, 128), jnp.float32),
        scratch_types=_sc_scratch(epc),
    )
    def agg_kernel(h_hbm, src2_hbm, dst_hbm, out_hbm, srcall, *scr):
        dstbs, rowsbs, acc, gs, ss, ds = _unpack_scr(scr)
        c = lax.axis_index("c")
        s = lax.axis_index("s")
        row0 = s * rows_pt
        pltpu.sync_copy(h_hbm.at[pl.ds(goff + c * _NP + row0, rows_pt)],
                        acc.at[pl.ds(row0, rows_pt)])
        plsc.subcore_barrier()
        _edge_pipeline(h_hbm, src2_hbm, c * _E + s * epc, dst_hbm, s * epc,
                       acc, srcall, dstbs, rowsbs, gs, ss, ds, epc)
        plsc.subcore_barrier()
        pltpu.sync_copy(acc.at[pl.ds(row0, rows_pt)],
                        out_hbm.at[pl.ds(c * _NP + row0, rows_pt)])

    return agg_kernel


def _make_sc_agg_graphs():
    mesh = plsc.VectorSubcoreMesh(core_axis_name="c", subcore_axis_name="s")
    epc = _E // _TILES
    rows_pt = _NP // _TILES

    @functools.partial(
        pl.kernel,
        mesh=mesh,
        out_type=jax.ShapeDtypeStruct((2 * _NP, 128), jnp.float32),
        scratch_types=_sc_scratch(epc),
    )
    def agg_kernel(xab_hbm, srcab_hbm, dstab_hbm, out_hbm, srcall, *scr):
        dstbs, rowsbs, acc, gs, ss, ds = _unpack_scr(scr)
        c = lax.axis_index("c")
        s = lax.axis_index("s")
        row0 = s * rows_pt
        pltpu.sync_copy(xab_hbm.at[pl.ds(c * _NP + row0, rows_pt)],
                        acc.at[pl.ds(row0, rows_pt)])
        plsc.subcore_barrier()
        base = c * _E + s * epc
        _edge_pipeline(xab_hbm, srcab_hbm, base, dstab_hbm, base,
                       acc, srcall, dstbs, rowsbs, gs, ss, ds, epc)
        plsc.subcore_barrier()
        pltpu.sync_copy(acc.at[pl.ds(row0, rows_pt)],
                        out_hbm.at[pl.ds(c * _NP + row0, rows_pt)])

    return agg_kernel


def _make_tc_layer1():
    def body(x_ref, w_ref, b_ref, o_ref):
        z = jnp.dot(x_ref[...], w_ref[...], preferred_element_type=jnp.float32)
        o_ref[...] = jnp.maximum(z + b_ref[...], 0.0)

    return pl.pallas_call(
        body,
        grid=(2, 2, _NB),
        in_specs=[
            pl.BlockSpec((_BN, 128), lambda g, h, i: (g * _NB + i, 0)),
            pl.BlockSpec((128, 128), lambda g, h, i: (0, h)),
            pl.BlockSpec((1, 128), lambda g, h, i: (0, h)),
        ],
        out_specs=pl.BlockSpec((_BN, 128),
                               lambda g, h, i: ((2 * g + h) * _NB + i, 0)),
        out_shape=jax.ShapeDtypeStruct((4 * _NP, 128), jnp.float32),
    )


def _make_tc_layer23():
    def body(x_lo_ref, x_hi_ref, w_ref, b_ref, o_ref):
        w = w_ref[...]
        z = (jnp.dot(x_lo_ref[...], w[:128, :],
                     preferred_element_type=jnp.float32)
             + jnp.dot(x_hi_ref[...], w[128:, :],
                       preferred_element_type=jnp.float32))
        o_ref[...] = jnp.maximum(z + b_ref[...], 0.0)

    return pl.pallas_call(
        body,
        grid=(2, _NB),
        in_specs=[
            pl.BlockSpec((_BN, 128), lambda h, i: (i, 0)),
            pl.BlockSpec((_BN, 128), lambda h, i: (i + _NB, 0)),
            pl.BlockSpec((256, 128), lambda h, i: (0, h)),
            pl.BlockSpec((1, 128), lambda h, i: (0, h)),
        ],
        out_specs=pl.BlockSpec((_BN, 128), lambda h, i: (h * _NB + i, 0)),
        out_shape=jax.ShapeDtypeStruct((2 * _NP, 128), jnp.float32),
    )


def _final_body(hA_lo, hA_hi, bA_ref, hB_lo, hB_hi, bB_ref, w_ref, b_ref,
                o_ref, sumsA, cntA, sumsB, cntB):
    i = pl.program_id(0)
    iota = lax.broadcasted_iota(jnp.int32, (_BN, _G), 1)
    ones_col = jnp.ones((_BN, 1), jnp.float32)
    dn = (((0,), (0,)), ((), ()))
    mA = (bA_ref[...] == iota).astype(jnp.float32)
    mB = (bB_ref[...] == iota).astype(jnp.float32)
    sA_lo = lax.dot_general(mA, hA_lo[...], dn, preferred_element_type=jnp.float32)
    sA_hi = lax.dot_general(mA, hA_hi[...], dn, preferred_element_type=jnp.float32)
    sB_lo = lax.dot_general(mB, hB_lo[...], dn, preferred_element_type=jnp.float32)
    sB_hi = lax.dot_general(mB, hB_hi[...], dn, preferred_element_type=jnp.float32)
    cA = lax.dot_general(mA, ones_col, dn, preferred_element_type=jnp.float32)
    cB = lax.dot_general(mB, ones_col, dn, preferred_element_type=jnp.float32)

    @pl.when(i == 0)
    def _():
        sumsA[:, :128] = sA_lo
        sumsA[:, 128:] = sA_hi
        sumsB[:, :128] = sB_lo
        sumsB[:, 128:] = sB_hi
        cntA[...] = cA
        cntB[...] = cB

    @pl.when(i > 0)
    def _():
        sumsA[:, :128] += sA_lo
        sumsA[:, 128:] += sA_hi
        sumsB[:, :128] += sB_lo
        sumsB[:, 128:] += sB_hi
        cntA[...] += cA
        cntB[...] += cB

    @pl.when(i == _NB - 1)
    def _():
        pooledA = sumsA[...] / jnp.maximum(cntA[...], 1.0)
        pooledB = sumsB[...] / jnp.maximum(cntB[...], 1.0)
        w = w_ref[...]
        b = b_ref[...]
        embA = jnp.dot(pooledA, w, preferred_element_type=jnp.float32) + b
        embB = jnp.dot(pooledB, w, preferred_element_type=jnp.float32) + b
        num = jnp.sum(embA * embB, axis=1, keepdims=True)
        nA = jnp.sqrt(jnp.sum(embA * embA, axis=1, keepdims=True))
        nB = jnp.sqrt(jnp.sum(embB * embB, axis=1, keepdims=True))
        o_ref[...] = num / jnp.maximum(nA * nB, 1e-8)


def _make_final():
    return pl.pallas_call(
        _final_body,
        grid=(_NB,),
        in_specs=[
            pl.BlockSpec((_BN, 128), lambda i: (i, 0)),
            pl.BlockSpec((_BN, 128), lambda i: (i + _NB, 0)),
            pl.BlockSpec((_BN, 1), lambda i: (i, 0)),
            pl.BlockSpec((_BN, 128), lambda i: (i, 0)),
            pl.BlockSpec((_BN, 128), lambda i: (i + _NB, 0)),
            pl.BlockSpec((_BN, 1), lambda i: (i, 0)),
            pl.BlockSpec((256, 128), lambda i: (0, 0)),
            pl.BlockSpec((1, 128), lambda i: (0, 0)),
        ],
        out_specs=pl.BlockSpec((_G, 1), lambda i: (0, 0)),
        out_shape=jax.ShapeDtypeStruct((_G, 1), jnp.float32),
        scratch_shapes=[
            pltpu.VMEM((_G, 256), jnp.float32),
            pltpu.VMEM((_G, 1), jnp.float32),
            pltpu.VMEM((_G, 256), jnp.float32),
            pltpu.VMEM((_G, 1), jnp.float32),
        ],
    )


def kernel(x_A, edge_index_A, batch_A, x_B, edge_index_B, batch_B,
           W_in, b_in, W_h1, b_h1, W_h2, b_h2, W_out, b_out):
    agg_graphs = _make_sc_agg_graphs()
    agg_feat0 = _make_sc_agg_feat(0)
    agg_featB = _make_sc_agg_feat(2 * _NP)
    layer1 = _make_tc_layer1()
    layer23 = _make_tc_layer23()
    final = _make_final()

    b_in2 = b_in.astype(jnp.float32).reshape(1, 256)
    b_h12 = b_h1.astype(jnp.float32).reshape(1, 256)
    b_h22 = b_h2.astype(jnp.float32).reshape(1, 256)
    b_out2 = b_out.astype(jnp.float32).reshape(1, 128)
    pad = jnp.zeros((_NP - _N, 128), jnp.float32)

    srcA = edge_index_A[0].astype(jnp.int32)
    dstA = edge_index_A[1].astype(jnp.int32)
    srcB = edge_index_B[0].astype(jnp.int32)
    dstB = edge_index_B[1].astype(jnp.int32)
    src2A = jnp.concatenate([srcA, srcA + _NP])
    src2B = jnp.concatenate([srcB, srcB + _NP])

    xab = jnp.concatenate([x_A, pad, x_B, pad])
    srcab = jnp.concatenate([srcA, srcB + _NP])
    dstab = jnp.concatenate([dstA, dstB])
    a1 = agg_graphs(xab, srcab, dstab)
    h1ab = layer1(a1, W_in, b_in2)

    a2A = agg_feat0(h1ab, src2A, dstA)
    a2B = agg_featB(h1ab, src2B + 2 * _NP, dstB)
    h2A = layer23(a2A, a2A, W_h1, b_h12)
    h2B = layer23(a2B, a2B, W_h1, b_h12)
    a3A = agg_feat0(h2A, src2A, dstA)
    a3B = agg_feat0(h2B, src2B, dstB)
    h3A = layer23(a3A, a3A, W_h2, b_h22)
    h3B = layer23(a3B, a3B, W_h2, b_h22)

    def pad_batch(batch):
        b = jnp.concatenate([batch.astype(jnp.int32),
                             jnp.full((_NP - _N,), -1, jnp.int32)])
        return b.reshape(_NP, 1)

    score = final(h3A, h3A, pad_batch(batch_A),
                  h3B, h3B, pad_batch(batch_B),
                  W_out, b_out2)
    return score.reshape(_G)


# speedup vs baseline: 1.0539x; 1.0539x over previous
import functools

import jax
import jax.numpy as jnp
from jax import lax
from jax.experimental import pallas as pl
from jax.experimental.pallas import tpu as pltpu
from jax.experimental.pallas import tpu_sc as plsc

_N = 10000
_NP = 10240
_E = 320000
_G = 128
_TILES = 16
_BN = 640
_NB = _NP // _BN
_R = 5
_C = 40


def _edge_pipeline(table_hbm, src_hbm, src_base, dst_hbm, dst_base, acc,
                   srcall, dstbs, rowsbs, gs, ss, ds, epc, init_wait):
    n = epc // _C
    nmac = n // _R
    pltpu.sync_copy(src_hbm.at[pl.ds(src_base, epc)], srcall)

    def src_sl(k):
        return srcall.at[pl.ds(k * _C, _C)]

    def g_start(k, p):
        pltpu.async_copy(table_hbm.at[src_sl(k)], rowsbs[p], gs[p])

    def g_wait(k, p):
        pltpu.make_async_copy(table_hbm.at[src_sl(k)], rowsbs[p], gs[p]).wait()

    def d_start(k, p):
        pltpu.async_copy(dst_hbm.at[pl.ds(dst_base + k * _C, _C)], dstbs[p], ds[p])

    def d_wait(k, p):
        pltpu.make_async_copy(dst_hbm.at[pl.ds(dst_base + k * _C, _C)],
                              dstbs[p], ds[p]).wait()

    def s_start(p):
        pltpu.async_copy(rowsbs[p], acc.at[dstbs[p]], ss[p], add=True)

    def s_wait(p):
        pltpu.make_async_copy(rowsbs[p], acc.at[dstbs[p]], ss[p]).wait()

    for j in range(_R - 1):
        d_start(j, j)
        g_start(j, j)

    init_wait()
    plsc.subcore_barrier()

    def body(m, carry):
        k0 = m * _R
        for t in range(_R):
            k = k0 + t
            q = (t + _R - 1) % _R
            g_wait(k, t)
            d_wait(k, t)
            s_start(t)
            if t == 0:
                @pl.when(m > 0)
                def _():
                    s_wait(q)
            else:
                s_wait(q)

            @pl.when(k + _R - 1 < n)
            def _():
                d_start(k + _R - 1, q)
                g_start(k + _R - 1, q)

        return carry

    lax.fori_loop(0, nmac, body, 0)
    s_wait((_R - 1) % _R)


def _sc_scratch(epc):
    return ([pltpu.VMEM((epc,), jnp.int32)]
            + [pltpu.VMEM((_C,), jnp.int32) for _ in range(_R)]
            + [pltpu.VMEM((_C, 128), jnp.float32) for _ in range(_R)]
            + [pltpu.VMEM_SHARED((_NP, 128), jnp.float32)]
            + [pltpu.SemaphoreType.DMA for _ in range(3 * _R + 1)])


def _unpack_scr(scr):
    dstbs = scr[:_R]
    rowsbs = scr[_R:2 * _R]
    acc = scr[2 * _R]
    gs = scr[2 * _R + 1:3 * _R + 1]
    ss = scr[3 * _R + 1:4 * _R + 1]
    ds = scr[4 * _R + 1:5 * _R + 1]
    semi = scr[5 * _R + 1]
    return dstbs, rowsbs, acc, gs, ss, ds, semi


def _make_sc_agg_feat():
    mesh = plsc.VectorSubcoreMesh(core_axis_name="c", subcore_axis_name="s")
    epc = _E // _TILES
    rows_pt = _NP // _TILES

    @functools.partial(
        pl.kernel,
        mesh=mesh,
        out_type=jax.ShapeDtypeStruct((2 * _NP, 128), jnp.float32),
        scratch_types=_sc_scratch(epc),
    )
    def agg_kernel(h_hbm, src2_hbm, dst_hbm, out_hbm, srcall, *scr):
        dstbs, rowsbs, acc, gs, ss, ds, semi = _unpack_scr(scr)
        c = lax.axis_index("c")
        s = lax.axis_index("s")
        row0 = s * rows_pt
        pltpu.async_copy(h_hbm.at[pl.ds(c * _NP + row0, rows_pt)],
                         acc.at[pl.ds(row0, rows_pt)], semi)

        def init_wait():
            pltpu.make_async_copy(h_hbm.at[pl.ds(c * _NP + row0, rows_pt)],
                                  acc.at[pl.ds(row0, rows_pt)], semi).wait()

        _edge_pipeline(h_hbm, src2_hbm, c * _E + s * epc, dst_hbm, s * epc,
                       acc, srcall, dstbs, rowsbs, gs, ss, ds, epc, init_wait)
        plsc.subcore_barrier()
        pltpu.sync_copy(acc.at[pl.ds(row0, rows_pt)],
                        out_hbm.at[pl.ds(c * _NP + row0, rows_pt)])

    return agg_kernel


def _make_sc_agg_edge():
    mesh = plsc.VectorSubcoreMesh(core_axis_name="c", subcore_axis_name="s")
    e_half = _E // 2
    epc = e_half // _TILES
    rows_pt = _NP // _TILES

    @functools.partial(
        pl.kernel,
        mesh=mesh,
        out_type=jax.ShapeDtypeStruct((2 * _NP, 128), jnp.float32),
        scratch_types=_sc_scratch(epc),
    )
    def agg_kernel(x_hbm, zeros_hbm, src_hbm, dst_hbm, out_hbm, srcall, *scr):
        dstbs, rowsbs, acc, gs, ss, ds, semi = _unpack_scr(scr)
        c = lax.axis_index("c")
        s = lax.axis_index("s")
        row0 = s * rows_pt

        @pl.when(c == 0)
        def _():
            pltpu.async_copy(x_hbm.at[pl.ds(row0, rows_pt)],
                             acc.at[pl.ds(row0, rows_pt)], semi)

        @pl.when(c == 1)
        def _():
            pltpu.async_copy(zeros_hbm, acc.at[pl.ds(row0, rows_pt)], semi)

        def init_wait():
            pltpu.make_async_copy(x_hbm.at[pl.ds(row0, rows_pt)],
                                  acc.at[pl.ds(row0, rows_pt)], semi).wait()

        base = c * e_half + s * epc
        _edge_pipeline(x_hbm, src_hbm, base, dst_hbm, base,
                       acc, srcall, dstbs, rowsbs, gs, ss, ds, epc, init_wait)
        plsc.subcore_barrier()
        pltpu.sync_copy(acc.at[pl.ds(row0, rows_pt)],
                        out_hbm.at[pl.ds(c * _NP + row0, rows_pt)])

    return agg_kernel


def _make_tc_layer1():
    def body(x_lo_ref, x_hi_ref, w_ref, b_ref, o_ref):
        z = jnp.dot(x_lo_ref[...] + x_hi_ref[...], w_ref[...],
                    preferred_element_type=jnp.float32)
        o_ref[...] = jnp.maximum(z + b_ref[...], 0.0)

    return pl.pallas_call(
        body,
        grid=(2, _NB),
        in_specs=[
            pl.BlockSpec((_BN, 128), lambda h, i: (i, 0)),
            pl.BlockSpec((_BN, 128), lambda h, i: (i + _NB, 0)),
            pl.BlockSpec((128, 128), lambda h, i: (0, h)),
            pl.BlockSpec((1, 128), lambda h, i: (0, h)),
        ],
        out_specs=pl.BlockSpec((_BN, 128), lambda h, i: (h * _NB + i, 0)),
        out_shape=jax.ShapeDtypeStruct((2 * _NP, 128), jnp.float32),
    )


def _make_tc_layer23():
    def body(x_lo_ref, x_hi_ref, w_ref, b_ref, o_ref):
        w = w_ref[...]
        z = (jnp.dot(x_lo_ref[...], w[:128, :],
                     preferred_element_type=jnp.float32)
             + jnp.dot(x_hi_ref[...], w[128:, :],
                       preferred_element_type=jnp.float32))
        o_ref[...] = jnp.maximum(z + b_ref[...], 0.0)

    return pl.pallas_call(
        body,
        grid=(2, _NB),
        in_specs=[
            pl.BlockSpec((_BN, 128), lambda h, i: (i, 0)),
            pl.BlockSpec((_BN, 128), lambda h, i: (i + _NB, 0)),
            pl.BlockSpec((256, 128), lambda h, i: (0, h)),
            pl.BlockSpec((1, 128), lambda h, i: (0, h)),
        ],
        out_specs=pl.BlockSpec((_BN, 128), lambda h, i: (h * _NB + i, 0)),
        out_shape=jax.ShapeDtypeStruct((2 * _NP, 128), jnp.float32),
    )


def _final_body(hA_lo, hA_hi, bA_ref, hB_lo, hB_hi, bB_ref, w_ref, b_ref,
                o_ref, sumsA, cntA, sumsB, cntB):
    i = pl.program_id(0)
    iota = lax.broadcasted_iota(jnp.int32, (_BN, _G), 1)
    ones_col = jnp.ones((_BN, 1), jnp.float32)
    dn = (((0,), (0,)), ((), ()))
    mA = (bA_ref[...] == iota).astype(jnp.float32)
    mB = (bB_ref[...] == iota).astype(jnp.float32)
    sA_lo = lax.dot_general(mA, hA_lo[...], dn, preferred_element_type=jnp.float32)
    sA_hi = lax.dot_general(mA, hA_hi[...], dn, preferred_element_type=jnp.float32)
    sB_lo = lax.dot_general(mB, hB_lo[...], dn, preferred_element_type=jnp.float32)
    sB_hi = lax.dot_general(mB, hB_hi[...], dn, preferred_element_type=jnp.float32)
    cA = lax.dot_general(mA, ones_col, dn, preferred_element_type=jnp.float32)
    cB = lax.dot_general(mB, ones_col, dn, preferred_element_type=jnp.float32)

    @pl.when(i == 0)
    def _():
        sumsA[:, :128] = sA_lo
        sumsA[:, 128:] = sA_hi
        sumsB[:, :128] = sB_lo
        sumsB[:, 128:] = sB_hi
        cntA[...] = cA
        cntB[...] = cB

    @pl.when(i > 0)
    def _():
        sumsA[:, :128] += sA_lo
        sumsA[:, 128:] += sA_hi
        sumsB[:, :128] += sB_lo
        sumsB[:, 128:] += sB_hi
        cntA[...] += cA
        cntB[...] += cB

    @pl.when(i == _NB - 1)
    def _():
        pooledA = sumsA[...] / jnp.maximum(cntA[...], 1.0)
        pooledB = sumsB[...] / jnp.maximum(cntB[...], 1.0)
        w = w_ref[...]
        b = b_ref[...]
        embA = jnp.dot(pooledA, w, preferred_element_type=jnp.float32) + b
        embB = jnp.dot(pooledB, w, preferred_element_type=jnp.float32) + b
        num = jnp.sum(embA * embB, axis=1, keepdims=True)
        nA = jnp.sqrt(jnp.sum(embA * embA, axis=1, keepdims=True))
        nB = jnp.sqrt(jnp.sum(embB * embB, axis=1, keepdims=True))
        o_ref[...] = num / jnp.maximum(nA * nB, 1e-8)


def _make_final():
    return pl.pallas_call(
        _final_body,
        grid=(_NB,),
        in_specs=[
            pl.BlockSpec((_BN, 128), lambda i: (i, 0)),
            pl.BlockSpec((_BN, 128), lambda i: (i + _NB, 0)),
            pl.BlockSpec((_BN, 1), lambda i: (i, 0)),
            pl.BlockSpec((_BN, 128), lambda i: (i, 0)),
            pl.BlockSpec((_BN, 128), lambda i: (i + _NB, 0)),
            pl.BlockSpec((_BN, 1), lambda i: (i, 0)),
            pl.BlockSpec((256, 128), lambda i: (0, 0)),
            pl.BlockSpec((1, 128), lambda i: (0, 0)),
        ],
        out_specs=pl.BlockSpec((_G, 1), lambda i: (0, 0)),
        out_shape=jax.ShapeDtypeStruct((_G, 1), jnp.float32),
        scratch_shapes=[
            pltpu.VMEM((_G, 256), jnp.float32),
            pltpu.VMEM((_G, 1), jnp.float32),
            pltpu.VMEM((_G, 256), jnp.float32),
            pltpu.VMEM((_G, 1), jnp.float32),
        ],
    )


def kernel(x_A, edge_index_A, batch_A, x_B, edge_index_B, batch_B,
           W_in, b_in, W_h1, b_h1, W_h2, b_h2, W_out, b_out):
    agg_edge = _make_sc_agg_edge()
    agg_feat = _make_sc_agg_feat()
    layer1 = _make_tc_layer1()
    layer23 = _make_tc_layer23()
    final = _make_final()

    b_in2 = b_in.astype(jnp.float32).reshape(1, 256)
    b_h12 = b_h1.astype(jnp.float32).reshape(1, 256)
    b_h22 = b_h2.astype(jnp.float32).reshape(1, 256)
    b_out2 = b_out.astype(jnp.float32).reshape(1, 128)
    pad = jnp.zeros((_NP - _N, 128), jnp.float32)
    zeros_pt = jnp.zeros((_NP // _TILES, 128), jnp.float32)

    srcA = edge_index_A[0].astype(jnp.int32)
    dstA = edge_index_A[1].astype(jnp.int32)
    srcB = edge_index_B[0].astype(jnp.int32)
    dstB = edge_index_B[1].astype(jnp.int32)
    src2A = jnp.concatenate([srcA, srcA + _NP])
    src2B = jnp.concatenate([srcB, srcB + _NP])

    a1A = agg_edge(jnp.concatenate([x_A, pad]), zeros_pt, srcA, dstA)
    a1B = agg_edge(jnp.concatenate([x_B, pad]), zeros_pt, srcB, dstB)
    h1A = layer1(a1A, a1A, W_in, b_in2)
    h1B = layer1(a1B, a1B, W_in, b_in2)
    a2A = agg_feat(h1A, src2A, dstA)
    a2B = agg_feat(h1B, src2B, dstB)
    h2A = layer23(a2A, a2A, W_h1, b_h12)
    h2B = layer23(a2B, a2B, W_h1, b_h12)
    a3A = agg_feat(h2A, src2A, dstA)
    a3B = agg_feat(h2B, src2B, dstB)
    h3A = layer23(a3A, a3A, W_h2, b_h22)
    h3B = layer23(a3B, a3B, W_h2, b_h22)

    def pad_batch(batch):
        b = jnp.concatenate([batch.astype(jnp.int32),
                             jnp.full((_NP - _N,), -1, jnp.int32)])
        return b.reshape(_NP, 1)

    score = final(h3A, h3A, pad_batch(batch_A),
                  h3B, h3B, pad_batch(batch_B),
                  W_out, b_out2)
    return score.reshape(_G)
